# baseline (speedup 1.0000x reference)
import jax
import jax.numpy as jnp
from jax.experimental import pallas as pl
from jax.experimental.pallas import tpu as pltpu

_NP = 1280
_CHUNK = 32
_MAX_CHUNKS = 8
_ALIGN = 16


def _moe_body(ids_s_ref, ids_ref, wts_ref, x_ref, w13_ref, w2_ref,
              out_ref, xs_ref, os_ref, p_ref, start_s, nch_s):
    e = pl.program_id(0)
    nexp = pl.num_programs(0)
    tokens = x_ref.shape[0]
    topk = ids_ref.shape[1]

    @pl.when(e == 0)
    def _prep():
        def _zero(i, _):
            nch_s[i] = 0
            return 0
        jax.lax.fori_loop(0, nexp, _zero, 0)

        def _count(s, _):
            ee = ids_s_ref[s // topk, s % topk]
            nch_s[ee] = nch_s[ee] + 1
            return 0
        jax.lax.fori_loop(0, tokens * topk, _count, 0)

        def _starts(ee, run):
            c = nch_s[ee]
            start_s[ee] = run
            nch_s[ee] = (c + _CHUNK - 1) // _CHUNK
            return run + ((c + _ALIGN - 1) // _ALIGN) * _ALIGN
        jax.lax.fori_loop(0, nexp, _starts, 0)

        ids = ids_ref[...]
        wts = wts_ref[...]
        ecols = jax.lax.broadcasted_iota(jnp.int32, (tokens, nexp), 1)
        oh0 = (ids[:, 0:1] == ecols).astype(jnp.float32)
        oh1 = (ids[:, 1:2] == ecols).astype(jnp.float32)
        ctok = oh0 + oh1

        rr = jax.lax.broadcasted_iota(jnp.int32, (tokens, tokens), 0)
        cc = jax.lax.broadcasted_iota(jnp.int32, (tokens, tokens), 1)
        tril = (rr > cc).astype(jnp.float32)
        count_lt = jax.lax.dot_general(
            tril, ctok, (((1,), (0,)), ((), ())),
            preferred_element_type=jnp.float32)

        cnt = jnp.sum(ctok, axis=0, keepdims=True)
        pad = (((cnt.astype(jnp.int32) + _ALIGN - 1) // _ALIGN)
               * _ALIGN).astype(jnp.float32)
        er = jax.lax.broadcasted_iota(jnp.int32, (nexp, nexp), 0)
        ec = jax.lax.broadcasted_iota(jnp.int32, (nexp, nexp), 1)
        before = (er < ec).astype(jnp.float32)
        start = jax.lax.dot_general(
            pad, before, (((1,), (0,)), ((), ())),
            preferred_element_type=jnp.float32)

        rank0 = jnp.sum(count_lt * oh0, axis=1, keepdims=True)
        rank1 = (jnp.sum(count_lt * oh1, axis=1, keepdims=True)
                 + (ids[:, 0:1] == ids[:, 1:2]).astype(jnp.float32))
        pos0 = (jnp.sum(oh0 * start, axis=1, keepdims=True)
                + rank0).astype(jnp.int32)
        pos1 = (jnp.sum(oh1 * start, axis=1, keepdims=True)
                + rank1).astype(jnp.int32)

        piota = jax.lax.broadcasted_iota(jnp.int32, (tokens, _NP), 1)
        is0 = (piota == pos0).astype(jnp.float32)
        is1 = (piota == pos1).astype(jnp.float32)
        p_ref[...] = wts[:, 0:1] * is0 + wts[:, 1:2] * is1
        gt = (is0 + is1).astype(jnp.bfloat16)

        os_ref[...] = jnp.zeros_like(os_ref)
        xs_ref[...] = jax.lax.dot_general(
            gt, x_ref[...], (((0,), (0,)), ((), ())),
            preferred_element_type=jnp.float32).astype(jnp.bfloat16)

    w13 = w13_ref[0].astype(jnp.bfloat16)
    w2 = w2_ref[0].astype(jnp.bfloat16)
    inter = w2.shape[1]
    base = start_s[e]

    for c in range(_MAX_CHUNKS):
        @pl.when(c < nch_s[e])
        def _chunk(c=c):
            row = pl.multiple_of(base + c * _CHUNK, _ALIGN)
            xc = xs_ref[pl.ds(row, _CHUNK), :]
            gu = jax.lax.dot_general(
                xc, w13, (((1,), (1,)), ((), ())),
                preferred_element_type=jnp.float32)
            gate = gu[:, :inter]
            up = gu[:, inter:]
            h = (gate * jax.nn.sigmoid(gate) * up).astype(jnp.bfloat16)
            o = jax.lax.dot_general(
                h, w2, (((1,), (1,)), ((), ())),
                preferred_element_type=jnp.float32)
            os_ref[pl.ds(row, _CHUNK), :] = o

    @pl.when(e == nexp - 1)
    def _combine():
        out_ref[...] = jax.lax.dot_general(
            p_ref[...], os_ref[...], (((1,), (0,)), ((), ())),
            preferred_element_type=jnp.float32,
            precision=jax.lax.Precision.HIGHEST)


def kernel(hidden_states, topk_ids, topk_weights, w13, w2):
    tokens, hidden = hidden_states.shape
    num_experts, two_inter, _ = w13.shape
    inter = w2.shape[2]

    ids32 = topk_ids.astype(jnp.int32)
    wts = topk_weights.astype(jnp.float32)
    x16 = hidden_states.astype(jnp.bfloat16)

    out = pl.pallas_call(
        _moe_body,
        grid=(num_experts,),
        in_specs=[
            pl.BlockSpec(memory_space=pltpu.SMEM),
            pl.BlockSpec(ids32.shape, lambda e: (0, 0)),
            pl.BlockSpec(wts.shape, lambda e: (0, 0)),
            pl.BlockSpec((tokens, hidden), lambda e: (0, 0)),
            pl.BlockSpec((1, two_inter, hidden), lambda e: (e, 0, 0)),
            pl.BlockSpec((1, hidden, inter), lambda e: (e, 0, 0)),
        ],
        out_specs=pl.BlockSpec((tokens, hidden), lambda e: (0, 0)),
        out_shape=jax.ShapeDtypeStruct((tokens, hidden), jnp.float32),
        scratch_shapes=[
            pltpu.VMEM((_NP, hidden), jnp.bfloat16),
            pltpu.VMEM((_NP, hidden), jnp.float32),
            pltpu.VMEM((tokens, _NP), jnp.float32),
            pltpu.SMEM((num_experts,), jnp.int32),
            pltpu.SMEM((num_experts,), jnp.int32),
        ],
    )(ids32, ids32, wts, x16, w13, w2)
    return out
